# Initial kernel scaffold; baseline (speedup 1.0000x reference)
import jax
import jax.numpy as jnp
from jax.experimental import pallas as pl


def kernel(items, positions, targets):
    raise NotImplementedError("write your pallas kernel here")


# speedup vs baseline: 25.2521x; 25.2521x over previous
import functools

import jax
import jax.numpy as jnp
from jax import lax
from jax.experimental import pallas as pl
from jax.experimental.pallas import tpu as pltpu
from jax.experimental.pallas import tpu_sc as plsc

_LANES = 16


def _build_sc_call(rows, n):
    info = plsc.get_sparse_core_info()
    num_cores, num_subcores = info.num_cores, info.num_subcores
    num_workers = num_cores * num_subcores
    rows_per_worker = rows // num_workers
    assert rows_per_worker * num_workers == rows

    mesh = plsc.VectorSubcoreMesh(core_axis_name="c", subcore_axis_name="s")

    @functools.partial(
        pl.kernel,
        out_type=jax.ShapeDtypeStruct((rows, n), jnp.float32),
        mesh=mesh,
        scratch_types=[
            pltpu.VMEM((2 * n,), jnp.float32),
            pltpu.VMEM((n,), jnp.float32),
            pltpu.VMEM((_LANES,), jnp.float32),
        ],
        compiler_params=pltpu.CompilerParams(needs_layout_passes=False),
    )
    def sc_shift(items_hbm, pos_hbm, out_hbm, buf, ob, posv):
        wid = lax.axis_index("s") * num_cores + lax.axis_index("c")
        base = wid * rows_per_worker

        pltpu.sync_copy(pos_hbm.at[pl.ds(base, rows_per_worker)], posv)
        idx = (posv[...] * jnp.float32(n)).astype(jnp.int32)
        starts = jnp.int32(n) - idx

        def _zero(j, carry):
            buf[pl.ds(j * _LANES, _LANES)] = jnp.zeros((_LANES,), jnp.float32)
            return carry

        lax.fori_loop(0, n // _LANES, _zero, 0)

        lanes = lax.iota(jnp.int32, _LANES)
        unroll = 8
        for k in range(rows_per_worker):
            row = base + k
            start = starts[k]
            pltpu.sync_copy(items_hbm.at[row], buf.at[pl.ds(n, n)])

            def _shift(j, carry):
                off = j * (_LANES * unroll)
                for u in range(unroll):
                    v = plsc.load_gather(
                        buf, [lanes + (start + off + u * _LANES)])
                    ob[pl.ds(pl.multiple_of(off + u * _LANES, _LANES),
                             _LANES)] = v
                return carry

            lax.fori_loop(0, n // (_LANES * unroll), _shift, 0)
            pltpu.sync_copy(ob, out_hbm.at[row])

    return sc_shift


def kernel(items, positions, targets):
    b, nc, n = items.shape
    rows = b * nc
    items_f = items.reshape(rows, n)
    pos_f = positions.reshape(rows)
    out = _build_sc_call(rows, n)(items_f, pos_f)
    return out.reshape(b, nc, n)


# speedup vs baseline: 31.9933x; 1.2670x over previous
import functools

import jax
import jax.numpy as jnp
from jax import lax
from jax.experimental import pallas as pl
from jax.experimental.pallas import tpu as pltpu
from jax.experimental.pallas import tpu_sc as plsc

_LANES = 16
_UNROLL = 8


def _build_sc_call(rows, n):
    info = plsc.get_sparse_core_info()
    num_cores, num_subcores = info.num_cores, info.num_subcores
    num_workers = num_cores * num_subcores
    rpw = rows // num_workers
    assert rpw * num_workers == rows
    half = n // 2
    chunks = n // _LANES
    ch = chunks // 2

    mesh = plsc.VectorSubcoreMesh(core_axis_name="c", subcore_axis_name="s")

    @functools.partial(
        pl.kernel,
        out_type=jax.ShapeDtypeStruct((rows, n), jnp.float32),
        mesh=mesh,
        scratch_types=[
            pltpu.VMEM((n,), jnp.float32),
            pltpu.VMEM((n,), jnp.float32),
            pltpu.VMEM((half,), jnp.float32),
            pltpu.VMEM((half,), jnp.float32),
            pltpu.VMEM((_LANES,), jnp.float32),
            pltpu.SemaphoreType.DMA,
            pltpu.SemaphoreType.DMA,
            pltpu.SemaphoreType.DMA,
            pltpu.SemaphoreType.DMA,
        ],
        compiler_params=pltpu.CompilerParams(needs_layout_passes=False),
    )
    def sc_shift(items_hbm, pos_hbm, out_hbm, rb0, rb1, ob0, ob1, posv,
                 si0, si1, so0, so1):
        wid = lax.axis_index("s") * num_cores + lax.axis_index("c")
        base = wid * rpw

        pltpu.sync_copy(pos_hbm.at[pl.ds(base, rpw)], posv)
        idxv = (posv[...] * jnp.float32(n)).astype(jnp.int32)

        rbs, sins = (rb0, rb1), (si0, si1)
        obs, souts = (ob0, ob1), (so0, so1)
        lanes = lax.iota(jnp.int32, _LANES)
        zero16 = jnp.zeros((_LANES,), jnp.float32)

        pltpu.async_copy(items_hbm.at[base], rbs[0], sins[0])

        for r in range(rpw):
            rb = rbs[r % 2]
            if r + 1 < rpw:
                pltpu.async_copy(items_hbm.at[base + r + 1],
                                 rbs[(r + 1) % 2], sins[(r + 1) % 2])
            pltpu.make_async_copy(items_hbm.at[base + r], rb,
                                  sins[r % 2]).wait()

            idx = idxv[r]
            cz = idx >> 4

            for h in range(2):
                ob = obs[h]
                h0, h1 = h * ch, (h + 1) * ch
                if r >= 1:
                    pltpu.make_async_copy(
                        ob, out_hbm.at[base + r - 1, pl.ds(h0 * _LANES, half)],
                        souts[h]).wait()

                zend = jnp.minimum(jnp.maximum(cz, h0), h1)

                def _z(j, c, ob=ob, h0=h0):
                    ob[pl.ds(pl.multiple_of((j - h0) * _LANES, _LANES),
                             _LANES)] = zero16
                    return c

                lax.fori_loop(h0, zend, _z, 0)

                @pl.when(jnp.logical_and(cz >= h0, cz < h1))
                def _(ob=ob, rb=rb, h0=h0, cz=cz, idx=idx):
                    iv = lanes + (cz * _LANES - idx)
                    m = iv >= 0
                    v = plsc.load_gather(rb, [jnp.maximum(iv, 0)], mask=m)
                    ob[pl.ds(pl.multiple_of((cz - h0) * _LANES, _LANES),
                             _LANES)] = jnp.where(m, v, 0.0)

                gstart = jnp.minimum(jnp.maximum(cz + 1, h0), h1)
                nfull = (h1 - gstart) // _UNROLL
                iv0 = lanes + (gstart * _LANES - idx)

                def _g(bnum, iv, ob=ob, rb=rb, h0=h0, gstart=gstart):
                    jb = gstart + bnum * _UNROLL
                    for u in range(_UNROLL):
                        v = plsc.load_gather(rb, [iv + u * _LANES])
                        ob[pl.ds(pl.multiple_of((jb + u - h0) * _LANES,
                                                _LANES), _LANES)] = v
                    return iv + _UNROLL * _LANES

                iv_end = lax.fori_loop(0, nfull, _g, iv0)

                def _gr(j, iv, ob=ob, rb=rb, h0=h0):
                    v = plsc.load_gather(rb, [iv])
                    ob[pl.ds(pl.multiple_of((j - h0) * _LANES, _LANES),
                             _LANES)] = v
                    return iv + _LANES

                lax.fori_loop(gstart + nfull * _UNROLL, h1, _gr, iv_end)

                pltpu.async_copy(
                    ob, out_hbm.at[base + r, pl.ds(h0 * _LANES, half)],
                    souts[h])

        for h in range(2):
            pltpu.make_async_copy(
                obs[h],
                out_hbm.at[base + rpw - 1, pl.ds(h * half, half)],
                souts[h]).wait()

    return sc_shift


def kernel(items, positions, targets):
    b, nc, n = items.shape
    rows = b * nc
    items_f = items.reshape(rows, n)
    pos_f = positions.reshape(rows)
    out = _build_sc_call(rows, n)(items_f, pos_f)
    return out.reshape(b, nc, n)


# speedup vs baseline: 40.9467x; 1.2799x over previous
import functools

import jax
import jax.numpy as jnp
from jax import lax
from jax.experimental import pallas as pl
from jax.experimental.pallas import tpu as pltpu
from jax.experimental.pallas import tpu_sc as plsc

_LANES = 16
_UNROLL = 8


def _build_sc_call(rows, n):
    info = plsc.get_sparse_core_info()
    num_cores, num_subcores = info.num_cores, info.num_subcores
    num_workers = num_cores * num_subcores
    rpw = rows // num_workers
    assert rpw * num_workers == rows
    half = n // 2
    chunks = n // _LANES
    ch = chunks // 2

    mesh = plsc.VectorSubcoreMesh(core_axis_name="c", subcore_axis_name="s")

    @functools.partial(
        pl.kernel,
        out_type=jax.ShapeDtypeStruct((rows, n), jnp.float32),
        mesh=mesh,
        scratch_types=[
            pltpu.VMEM((n,), jnp.float32),
            pltpu.VMEM((n,), jnp.float32),
            pltpu.VMEM((half,), jnp.float32),
            pltpu.VMEM((half,), jnp.float32),
            pltpu.VMEM((_LANES,), jnp.float32),
            pltpu.SemaphoreType.DMA,
            pltpu.SemaphoreType.DMA,
            pltpu.SemaphoreType.DMA,
            pltpu.SemaphoreType.DMA,
        ],
        compiler_params=pltpu.CompilerParams(needs_layout_passes=False),
    )
    def sc_shift(items_hbm, pos_hbm, out_hbm, rb0, rb1, ob0, ob1, posv,
                 si0, si1, so0, so1):
        wid = lax.axis_index("s") * num_cores + lax.axis_index("c")
        base = wid * rpw

        pltpu.sync_copy(pos_hbm.at[pl.ds(base, rpw)], posv)
        idxv = (posv[...] * jnp.float32(n)).astype(jnp.int32)

        rbs, sins = (rb0, rb1), (si0, si1)
        obs, souts = (ob0, ob1), (so0, so1)
        lanes = lax.iota(jnp.int32, _LANES)
        zero16 = jnp.zeros((_LANES,), jnp.float32)

        pltpu.async_copy(items_hbm.at[base], rbs[0], sins[0])

        for r in range(rpw):
            rb = rbs[r % 2]
            if r + 1 < rpw:
                pltpu.async_copy(items_hbm.at[base + r + 1],
                                 rbs[(r + 1) % 2], sins[(r + 1) % 2])
            pltpu.make_async_copy(items_hbm.at[base + r], rb,
                                  sins[r % 2]).wait()

            idx = idxv[r]
            cz = idx >> 4

            for h in range(2):
                ob = obs[h]
                h0, h1 = h * ch, (h + 1) * ch
                if r >= 1:
                    pltpu.make_async_copy(
                        ob, out_hbm.at[base + r - 1, pl.ds(h0 * _LANES, half)],
                        souts[h]).wait()

                zend = jnp.minimum(jnp.maximum(cz, h0), h1)

                def _z(j, c, ob=ob, h0=h0):
                    ob[pl.ds(pl.multiple_of((j - h0) * _LANES, _LANES),
                             _LANES)] = zero16
                    return c

                lax.fori_loop(h0, zend, _z, 0)

                @pl.when(jnp.logical_and(cz >= h0, cz < h1))
                def _(ob=ob, rb=rb, h0=h0, cz=cz, idx=idx):
                    iv = lanes + (cz * _LANES - idx)
                    m = iv >= 0
                    v = plsc.load_gather(rb, [jnp.maximum(iv, 0)], mask=m)
                    ob[pl.ds(pl.multiple_of((cz - h0) * _LANES, _LANES),
                             _LANES)] = jnp.where(m, v, 0.0)

                gstart = jnp.minimum(jnp.maximum(cz + 1, h0), h1)
                nfull = (h1 - gstart) // _UNROLL

                def _loads(iv, rb=rb):
                    return tuple(plsc.load_gather(rb, [iv + u * _LANES])
                                 for u in range(_UNROLL))

                def _stores(jb, vs, ob=ob, h0=h0):
                    for u in range(_UNROLL):
                        ob[pl.ds(pl.multiple_of((jb + u - h0) * _LANES,
                                                _LANES), _LANES)] = vs[u]

                iv0 = lanes + (gstart * _LANES - idx)

                @pl.when(nfull > 0)
                def _(gstart=gstart, nfull=nfull, iv0=iv0,
                      _loads=_loads, _stores=_stores):
                    def _g(bnum, carry):
                        iv, prev = carry
                        iv_next = iv + _UNROLL * _LANES
                        cur = _loads(iv_next)
                        _stores(gstart + bnum * _UNROLL, prev)
                        return (iv_next, cur)

                    _, last = lax.fori_loop(0, nfull - 1, _g,
                                            (iv0, _loads(iv0)))
                    _stores(gstart + (nfull - 1) * _UNROLL, last)

                def _gr(j, iv, ob=ob, rb=rb, h0=h0):
                    v = plsc.load_gather(rb, [iv])
                    ob[pl.ds(pl.multiple_of((j - h0) * _LANES, _LANES),
                             _LANES)] = v
                    return iv + _LANES

                lax.fori_loop(gstart + nfull * _UNROLL, h1, _gr,
                              iv0 + nfull * (_UNROLL * _LANES))

                pltpu.async_copy(
                    ob, out_hbm.at[base + r, pl.ds(h0 * _LANES, half)],
                    souts[h])

        for h in range(2):
            pltpu.make_async_copy(
                obs[h],
                out_hbm.at[base + rpw - 1, pl.ds(h * half, half)],
                souts[h]).wait()

    return sc_shift


def kernel(items, positions, targets):
    b, nc, n = items.shape
    rows = b * nc
    items_f = items.reshape(rows, n)
    pos_f = positions.reshape(rows)
    out = _build_sc_call(rows, n)(items_f, pos_f)
    return out.reshape(b, nc, n)


# speedup vs baseline: 58.5007x; 1.4287x over previous
import functools

import jax
import jax.numpy as jnp
from jax import lax
from jax.experimental import pallas as pl
from jax.experimental.pallas import tpu as pltpu
from jax.experimental.pallas import tpu_sc as plsc

_LANES = 16
_UNROLL = 8
_BS = 8192


def _build_sc_call(rows, n):
    info = plsc.get_sparse_core_info()
    num_cores, num_subcores = info.num_cores, info.num_subcores
    num_workers = num_cores * num_subcores
    rpw = rows // num_workers
    assert rpw * num_workers == rows
    nb = n // _BS
    chunks = n // _LANES
    bs_chunks = _BS // _LANES

    mesh = plsc.VectorSubcoreMesh(core_axis_name="c", subcore_axis_name="s")

    @functools.partial(
        pl.kernel,
        out_type=jax.ShapeDtypeStruct((rows, n), jnp.float32),
        mesh=mesh,
        scratch_types=[
            pltpu.VMEM((n,), jnp.float32),
            pltpu.VMEM((n,), jnp.float32),
            pltpu.VMEM((n,), jnp.float32),
            pltpu.VMEM_SHARED((_BS,), jnp.float32),
            pltpu.VMEM((_LANES,), jnp.float32),
            pltpu.SemaphoreType.DMA,
            pltpu.SemaphoreType.DMA,
            pltpu.SemaphoreType.DMA,
            pltpu.SemaphoreType.DMA,
        ],
        compiler_params=pltpu.CompilerParams(needs_layout_passes=False),
    )
    def sc_shift(items_hbm, pos_hbm, out_hbm, rb0, rb1, ob, zsh, posv,
                 si0, si1, so, sz):
        wid = lax.axis_index("s") * num_cores + lax.axis_index("c")
        base = wid * rpw

        pltpu.sync_copy(pos_hbm.at[pl.ds(base, rpw)], posv)
        idxv = (posv[...] * jnp.float32(n)).astype(jnp.int32)

        lanes = lax.iota(jnp.int32, _LANES)
        zero16 = jnp.zeros((_LANES,), jnp.float32)
        rbs, sins = (rb0, rb1), (si0, si1)

        @pl.when(lax.axis_index("s") == 0)
        def _():
            def _z0(j, c):
                ob[pl.ds(j * _LANES, _LANES)] = zero16
                return c
            lax.fori_loop(0, bs_chunks, _z0, 0)
            pltpu.sync_copy(ob.at[pl.ds(0, _BS)], zsh)
        plsc.subcore_barrier()

        def _nb_in(idx):
            return (jnp.int32(n) - idx + jnp.int32(_BS - 1)) >> 13

        def _fire_in(r, idx):
            def _f(j, c, r=r):
                pltpu.async_copy(
                    items_hbm.at[base + r,
                                 pl.ds(pl.multiple_of(j * _BS, _BS), _BS)],
                    rbs[r % 2].at[pl.ds(pl.multiple_of(j * _BS, _BS), _BS)],
                    sins[r % 2])
                return c
            lax.fori_loop(0, _nb_in(idx), _f, 0)

        def _wait_in(r, idx):
            def _w(j, c, r=r):
                pltpu.make_async_copy(
                    items_hbm.at[base + r, pl.ds(0, _BS)],
                    rbs[r % 2].at[pl.ds(0, _BS)], sins[r % 2]).wait()
                return c
            lax.fori_loop(0, _nb_in(idx), _w, 0)

        def _drain_out(r, zb):
            def _w(j, c, r=r):
                pltpu.make_async_copy(
                    ob.at[pl.ds(0, _BS)],
                    out_hbm.at[base + r, pl.ds(0, _BS)], so).wait()
                return c
            lax.fori_loop(zb, nb, _w, 0)

        def _drain_zeros(zb):
            def _w(j, c):
                pltpu.make_async_copy(
                    zsh, out_hbm.at[base, pl.ds(0, _BS)], sz).wait()
                return c
            lax.fori_loop(0, zb, _w, 0)

        _fire_in(0, idxv[0])

        for r in range(rpw):
            idx = idxv[r]
            zb = idx >> 13
            cz = idx >> 4

            if r + 1 < rpw:
                _fire_in(r + 1, idxv[r + 1])
            _wait_in(r, idx)
            rb = rbs[r % 2]

            def _fz(j, c, r=r):
                pltpu.async_copy(
                    zsh,
                    out_hbm.at[base + r,
                               pl.ds(pl.multiple_of(j * _BS, _BS), _BS)],
                    sz)
                return c
            lax.fori_loop(0, zb, _fz, 0)

            if r >= 1:
                _drain_out(r - 1, idxv[r - 1] >> 13)

            def _pz(j, c):
                ob[pl.ds(pl.multiple_of(j * _LANES, _LANES),
                         _LANES)] = zero16
                return c
            lax.fori_loop(zb * bs_chunks, cz, _pz, 0)

            @pl.when(cz < chunks)
            def _(rb=rb, cz=cz, idx=idx):
                iv = lanes + (cz * _LANES - idx)
                m = iv >= 0
                v = plsc.load_gather(rb, [jnp.maximum(iv, 0)], mask=m)
                ob[pl.ds(pl.multiple_of(cz * _LANES, _LANES),
                         _LANES)] = jnp.where(m, v, 0.0)

            gstart = jnp.minimum(cz + 1, chunks)
            nfull = (chunks - gstart) // _UNROLL

            def _loads(iv, rb=rb):
                return tuple(plsc.load_gather(rb, [iv + u * _LANES])
                             for u in range(_UNROLL))

            def _stores(jb, vs):
                for u in range(_UNROLL):
                    ob[pl.ds(pl.multiple_of((jb + u) * _LANES, _LANES),
                             _LANES)] = vs[u]

            iv0 = lanes + (gstart * _LANES - idx)

            @pl.when(nfull > 0)
            def _(gstart=gstart, nfull=nfull, iv0=iv0,
                  _loads=_loads, _stores=_stores):
                def _g(bnum, carry):
                    iv, prev = carry
                    iv_next = iv + _UNROLL * _LANES
                    cur = _loads(iv_next)
                    _stores(gstart + bnum * _UNROLL, prev)
                    return (iv_next, cur)

                _, last = lax.fori_loop(0, nfull - 1, _g,
                                        (iv0, _loads(iv0)))
                _stores(gstart + (nfull - 1) * _UNROLL, last)

            def _gr(j, iv, rb=rb):
                v = plsc.load_gather(rb, [iv])
                ob[pl.ds(pl.multiple_of(j * _LANES, _LANES), _LANES)] = v
                return iv + _LANES

            lax.fori_loop(gstart + nfull * _UNROLL, chunks, _gr,
                          iv0 + nfull * (_UNROLL * _LANES))

            def _fo(j, c, r=r):
                pltpu.async_copy(
                    ob.at[pl.ds(pl.multiple_of(j * _BS, _BS), _BS)],
                    out_hbm.at[base + r,
                               pl.ds(pl.multiple_of(j * _BS, _BS), _BS)],
                    so)
                return c
            lax.fori_loop(zb, nb, _fo, 0)

        _drain_out(rpw - 1, idxv[rpw - 1] >> 13)
        for rr in range(rpw):
            _drain_zeros(idxv[rr] >> 13)

    return sc_shift


def kernel(items, positions, targets):
    b, nc, n = items.shape
    rows = b * nc
    items_f = items.reshape(rows, n)
    pos_f = positions.reshape(rows)
    out = _build_sc_call(rows, n)(items_f, pos_f)
    return out.reshape(b, nc, n)


# speedup vs baseline: 67.8825x; 1.1604x over previous
import functools

import jax
import jax.numpy as jnp
from jax import lax
from jax.experimental import pallas as pl
from jax.experimental.pallas import tpu as pltpu
from jax.experimental.pallas import tpu_sc as plsc

_LANES = 16
_UNROLL = 8
_BS = 8192


def _build_sc_call(rows, n):
    info = plsc.get_sparse_core_info()
    num_cores, num_subcores = info.num_cores, info.num_subcores
    num_workers = num_cores * num_subcores
    rpw = rows // num_workers
    assert rpw * num_workers == rows
    nb = n // _BS
    chunks = n // _LANES
    bs_chunks = _BS // _LANES

    mesh = plsc.VectorSubcoreMesh(core_axis_name="c", subcore_axis_name="s")

    @functools.partial(
        pl.kernel,
        out_type=jax.ShapeDtypeStruct((rows, n), jnp.float32),
        mesh=mesh,
        scratch_types=[
            pltpu.VMEM((n,), jnp.float32),
            pltpu.VMEM((n,), jnp.float32),
            pltpu.VMEM((n,), jnp.float32),
            pltpu.VMEM_SHARED((_BS,), jnp.float32),
            pltpu.VMEM((_LANES,), jnp.float32),
            pltpu.SemaphoreType.DMA,
            pltpu.SemaphoreType.DMA,
            pltpu.SemaphoreType.DMA,
            pltpu.SemaphoreType.DMA,
        ],
        compiler_params=pltpu.CompilerParams(needs_layout_passes=False),
    )
    def sc_shift(items_hbm, pos_hbm, out_hbm, rb0, rb1, ob, zsh, posv,
                 si0, si1, so, sz):
        wid = lax.axis_index("s") * num_cores + lax.axis_index("c")
        base = wid * rpw

        pltpu.sync_copy(pos_hbm.at[pl.ds(base, rpw)], posv)
        idxv = (posv[...] * jnp.float32(n)).astype(jnp.int32)

        lanes = lax.iota(jnp.int32, _LANES)
        zero16 = jnp.zeros((_LANES,), jnp.float32)
        rbs, sins = (rb0, rb1), (si0, si1)

        @pl.when(lax.axis_index("s") == 0)
        def _():
            def _z0(j, c):
                ob[pl.ds(j * _LANES, _LANES)] = zero16
                return c
            lax.fori_loop(0, bs_chunks, _z0, 0)
            pltpu.sync_copy(ob.at[pl.ds(0, _BS)], zsh)
        plsc.subcore_barrier()

        def _nb_in(idx):
            return (jnp.int32(n) - idx + jnp.int32(_BS - 1)) >> 13

        def _fire_in(r, idx):
            def _f(j, c, r=r):
                pltpu.async_copy(
                    items_hbm.at[base + r,
                                 pl.ds(pl.multiple_of(j * _BS, _BS), _BS)],
                    rbs[r % 2].at[pl.ds(pl.multiple_of(j * _BS, _BS), _BS)],
                    sins[r % 2])
                return c
            lax.fori_loop(0, _nb_in(idx), _f, 0)

        def _wait_in(r, idx):
            def _w(j, c, r=r):
                pltpu.make_async_copy(
                    items_hbm.at[base + r, pl.ds(0, _BS)],
                    rbs[r % 2].at[pl.ds(0, _BS)], sins[r % 2]).wait()
                return c
            lax.fori_loop(0, _nb_in(idx), _w, 0)

        def _drain_out(r, zb):
            def _w(j, c, r=r):
                pltpu.make_async_copy(
                    ob.at[pl.ds(0, _BS)],
                    out_hbm.at[base + r, pl.ds(0, _BS)], so).wait()
                return c
            lax.fori_loop(zb, nb, _w, 0)

        def _drain_zeros(zb):
            def _w(j, c):
                pltpu.make_async_copy(
                    zsh, out_hbm.at[base, pl.ds(0, _BS)], sz).wait()
                return c
            lax.fori_loop(0, zb, _w, 0)

        _fire_in(0, idxv[0])

        for r in range(rpw):
            idx = idxv[r]
            zb = idx >> 13
            cz = idx >> 4

            if r + 1 < rpw:
                _fire_in(r + 1, idxv[r + 1])
            _wait_in(r, idx)
            rb = rbs[r % 2]

            def _fz(j, c, r=r):
                pltpu.async_copy(
                    zsh,
                    out_hbm.at[base + r,
                               pl.ds(pl.multiple_of(j * _BS, _BS), _BS)],
                    sz)
                return c
            lax.fori_loop(0, zb, _fz, 0)

            if r >= 1:
                _drain_out(r - 1, idxv[r - 1] >> 13)

            pz0 = zb * bs_chunks
            pzfull = (cz - pz0) // _UNROLL

            def _pzb(bnum, c, pz0=pz0):
                jb = pz0 + bnum * _UNROLL
                for u in range(_UNROLL):
                    ob[pl.ds(pl.multiple_of((jb + u) * _LANES, _LANES),
                             _LANES)] = zero16
                return c
            lax.fori_loop(0, pzfull, _pzb, 0)

            def _pz(j, c):
                ob[pl.ds(pl.multiple_of(j * _LANES, _LANES),
                         _LANES)] = zero16
                return c
            lax.fori_loop(pz0 + pzfull * _UNROLL, cz, _pz, 0)

            @pl.when(cz < chunks)
            def _(rb=rb, cz=cz, idx=idx):
                iv = lanes + (cz * _LANES - idx)
                m = iv >= 0
                v = plsc.load_gather(rb, [jnp.maximum(iv, 0)], mask=m)
                ob[pl.ds(pl.multiple_of(cz * _LANES, _LANES),
                         _LANES)] = jnp.where(m, v, 0.0)

            gstart = jnp.minimum(cz + 1, chunks)
            nfull = (chunks - gstart) // _UNROLL

            def _loads(iv, rb=rb):
                return tuple(plsc.load_gather(rb, [iv + u * _LANES])
                             for u in range(_UNROLL))

            def _stores(jb, vs):
                for u in range(_UNROLL):
                    ob[pl.ds(pl.multiple_of((jb + u) * _LANES, _LANES),
                             _LANES)] = vs[u]

            iv0 = lanes + (gstart * _LANES - idx)

            @pl.when(nfull > 0)
            def _(gstart=gstart, nfull=nfull, iv0=iv0,
                  _loads=_loads, _stores=_stores):
                def _g(bnum, carry):
                    iv, prev = carry
                    iv_next = iv + _UNROLL * _LANES
                    cur = _loads(iv_next)
                    _stores(gstart + bnum * _UNROLL, prev)
                    return (iv_next, cur)

                _, last = lax.fori_loop(0, nfull - 1, _g,
                                        (iv0, _loads(iv0)))
                _stores(gstart + (nfull - 1) * _UNROLL, last)

            def _gr(j, iv, rb=rb):
                v = plsc.load_gather(rb, [iv])
                ob[pl.ds(pl.multiple_of(j * _LANES, _LANES), _LANES)] = v
                return iv + _LANES

            lax.fori_loop(gstart + nfull * _UNROLL, chunks, _gr,
                          iv0 + nfull * (_UNROLL * _LANES))

            def _fo(j, c, r=r):
                pltpu.async_copy(
                    ob.at[pl.ds(pl.multiple_of(j * _BS, _BS), _BS)],
                    out_hbm.at[base + r,
                               pl.ds(pl.multiple_of(j * _BS, _BS), _BS)],
                    so)
                return c
            lax.fori_loop(zb, nb, _fo, 0)

        _drain_out(rpw - 1, idxv[rpw - 1] >> 13)
        for rr in range(rpw):
            _drain_zeros(idxv[rr] >> 13)

    return sc_shift


def kernel(items, positions, targets):
    b, nc, n = items.shape
    rows = b * nc
    items_f = items.reshape(rows, n)
    pos_f = positions.reshape(rows)
    out = _build_sc_call(rows, n)(items_f, pos_f)
    return out.reshape(b, nc, n)
